# baseline (device time: 59188 ns/iter reference)
import jax
import jax.numpy as jnp
from jax import lax
from jax.experimental import pallas as pl
from jax.experimental.pallas import tpu as pltpu

N_DEV = 16
N_TOK = 2048
D_IN = 512
D_OUT = 1024
E_LOCAL = 4
CHUNK = N_TOK // N_DEV


def kernel(x, router_W, route_idx, expert_W, shared_W):
    expert_Wb = expert_W.astype(jnp.bfloat16)
    shared_Wb = shared_W.astype(jnp.bfloat16)

    def body(
        x_ref,
        rw_ref,
        ri_ref,
        ew_ref,
        sw_ref,
        out_ref,
        xb_ref,
        mp_ref,
        sb_ref,
        rb_ref,
        ssem,
        rsem,
    ):
        d = lax.axis_index("i")

        bsem = pltpu.get_barrier_semaphore()
        for off in range(1, N_DEV):
            pl.semaphore_signal(
                bsem, inc=1,
                device_id=((d + off) % N_DEV,),
                device_id_type=pl.DeviceIdType.MESH,
            )
        pl.semaphore_wait(bsem, N_DEV - 1)

        xb_ref[...] = x_ref[...].astype(jnp.bfloat16)
        scores = jnp.dot(x_ref[...], rw_ref[...],
                         preferred_element_type=jnp.float32)
        smax = jnp.max(scores, axis=-1, keepdims=True)
        p = jnp.exp(scores - smax)
        probs = p / jnp.sum(p, axis=-1, keepdims=True)
        iota = lax.broadcasted_iota(jnp.int32, (N_TOK, 64), 1)
        mp_ref[...] = jnp.where(iota == ri_ref[...], probs, 0.0)

        def partial_for(c):
            rows = pl.ds(c * CHUNK, CHUNK)
            xc = xb_ref[rows, :]
            mpc = mp_ref[rows, :]
            ci = lax.broadcasted_iota(jnp.int32, (CHUNK, 64), 1)
            acc = jnp.zeros((CHUNK, D_OUT), jnp.float32)
            for e in range(E_LOCAL):
                ge = d * E_LOCAL + e
                co = jnp.sum(jnp.where(ci == ge, mpc, 0.0),
                             axis=-1, keepdims=True)
                acc += jnp.dot(xc, ew_ref[e],
                               preferred_element_type=jnp.float32) * co
            return acc

        rdmas = []
        for off in range(1, N_DEV):
            dest = (d + off) % N_DEV
            sb_ref[off - 1] = partial_for(dest).astype(jnp.bfloat16)
            rdma = pltpu.make_async_remote_copy(
                src_ref=sb_ref.at[off - 1],
                dst_ref=rb_ref.at[N_DEV - 1 - off],
                send_sem=ssem.at[off - 1],
                recv_sem=rsem.at[N_DEV - 1 - off],
                device_id=(dest,),
                device_id_type=pl.DeviceIdType.MESH,
            )
            rdma.start()
            rdmas.append(rdma)

        own = partial_for(d)
        own += jnp.dot(xb_ref[pl.ds(d * CHUNK, CHUNK), :], sw_ref[...],
                       preferred_element_type=jnp.float32)

        for rdma in rdmas:
            rdma.wait()

        tot = own
        for i in range(N_DEV - 1):
            tot += rb_ref[i].astype(jnp.float32)
        out_ref[...] = tot

    return pl.pallas_call(
        body,
        out_shape=jax.ShapeDtypeStruct((CHUNK, D_OUT), jnp.float32),
        in_specs=[pl.BlockSpec(memory_space=pltpu.VMEM)] * 5,
        out_specs=pl.BlockSpec(memory_space=pltpu.VMEM),
        scratch_shapes=[
            pltpu.VMEM((N_TOK, D_IN), jnp.bfloat16),
            pltpu.VMEM((N_TOK, 64), jnp.float32),
            pltpu.VMEM((N_DEV - 1, CHUNK, D_OUT), jnp.bfloat16),
            pltpu.VMEM((N_DEV - 1, CHUNK, D_OUT), jnp.bfloat16),
            pltpu.SemaphoreType.DMA((N_DEV - 1,)),
            pltpu.SemaphoreType.DMA((N_DEV - 1,)),
        ],
        compiler_params=pltpu.CompilerParams(collective_id=0),
    )(x, router_W, route_idx, expert_Wb, shared_Wb)
